# Initial kernel scaffold; baseline (speedup 1.0000x reference)
import jax
import jax.numpy as jnp
from jax.experimental import pallas as pl


def kernel(input_ids, emb0, emb1, emb2, proj1_w, proj1_b, proj2_w, proj2_b):
    raise NotImplementedError("write your pallas kernel here")


# speedup vs baseline: 1.0073x; 1.0073x over previous
import functools

import jax
import jax.numpy as jnp
from jax import lax
from jax.experimental import pallas as pl
from jax.experimental.pallas import tpu as pltpu
from jax.experimental.pallas import tpu_sc as plsc

B, S, H = 4, 4096, 2048
N = B * S
D1, D2 = 512, 128
CUT0, CUT1 = 25000, 50000
SZ0, SZ1, SZ2 = 25000, 25000, 50000

NW = 32
TOK_PER_W = N // NW
CHUNK = 32
NCHUNK = TOK_PER_W // CHUNK

@functools.lru_cache(maxsize=1)
def _make_sc_gather():
    mesh = plsc.VectorSubcoreMesh(core_axis_name="c", subcore_axis_name="s")

    @functools.partial(
        pl.kernel,
        mesh=mesh,
        out_type=(
            jax.ShapeDtypeStruct((N, H), jnp.float32),
            jax.ShapeDtypeStruct((N, D1), jnp.float32),
            jax.ShapeDtypeStruct((N, D2), jnp.float32),
        ),
        scratch_types=[
            pltpu.VMEM((TOK_PER_W,), jnp.int32),
            pltpu.VMEM((TOK_PER_W,), jnp.int32),
            pltpu.VMEM((TOK_PER_W,), jnp.int32),
            pltpu.VMEM((TOK_PER_W,), jnp.int32),
            pltpu.VMEM((CHUNK, H), jnp.float32),
            pltpu.VMEM((CHUNK, D1), jnp.float32),
            pltpu.VMEM((CHUNK, D2), jnp.float32),
            pltpu.SemaphoreType.DMA,
        ],
    )
    def _sc_gather(ids_hbm, emb0_hbm, emb1_hbm, emb2_hbm,
                   g0_hbm, g1_hbm, g2_hbm,
                   ids_v, idx0_v, idx1_v, idx2_v, buf0, buf1, buf2, sem):
        wid = lax.axis_index("s") * 2 + lax.axis_index("c")
        base = wid * TOK_PER_W
        pltpu.sync_copy(ids_hbm.at[pl.ds(base, TOK_PER_W)], ids_v)
        for i in range(TOK_PER_W // 16):
            v = ids_v[pl.ds(i * 16, 16)]
            idx0_v[pl.ds(i * 16, 16)] = jnp.clip(v, 0, SZ0 - 1)
            idx1_v[pl.ds(i * 16, 16)] = jnp.clip(v - CUT0, 0, SZ1 - 1)
            idx2_v[pl.ds(i * 16, 16)] = jnp.clip(v - CUT1, 0, SZ2 - 1)
        for c in range(NCHUNK):
            off = c * CHUNK
            cp0 = pltpu.async_copy(emb0_hbm.at[idx0_v.at[pl.ds(off, CHUNK)]], buf0, sem)
            cp1 = pltpu.async_copy(emb1_hbm.at[idx1_v.at[pl.ds(off, CHUNK)]], buf1, sem)
            cp2 = pltpu.async_copy(emb2_hbm.at[idx2_v.at[pl.ds(off, CHUNK)]], buf2, sem)
            cp0.wait()
            cp1.wait()
            cp2.wait()
            pltpu.sync_copy(buf0, g0_hbm.at[pl.ds(base + off, CHUNK)])
            pltpu.sync_copy(buf1, g1_hbm.at[pl.ds(base + off, CHUNK)])
            pltpu.sync_copy(buf2, g2_hbm.at[pl.ds(base + off, CHUNK)])

    return _sc_gather


TB = 256


def _tc_body(ids_ref, g0_ref, g1_ref, g2_ref, w1_ref, b1_ref, w2_ref, b2_ref,
             out_ref):
    ids = ids_ref[...]
    p1 = lax.dot_general(g1_ref[...], w1_ref[...], (((1,), (1,)), ((), ())),
                         preferred_element_type=jnp.float32) + b1_ref[...]
    p2 = lax.dot_general(g2_ref[...], w2_ref[...], (((1,), (1,)), ((), ())),
                         preferred_element_type=jnp.float32) + b2_ref[...]
    p = jnp.where(ids < CUT1, p1, p2)
    out_ref[...] = jnp.where(ids < CUT0, g0_ref[...], p)


_tc_combine = pl.pallas_call(
    _tc_body,
    grid=(N // TB,),
    in_specs=[
        pl.BlockSpec((TB, 1), lambda i: (i, 0)),
        pl.BlockSpec((TB, H), lambda i: (i, 0)),
        pl.BlockSpec((TB, D1), lambda i: (i, 0)),
        pl.BlockSpec((TB, D2), lambda i: (i, 0)),
        pl.BlockSpec((H, D1), lambda i: (0, 0)),
        pl.BlockSpec((1, H), lambda i: (0, 0)),
        pl.BlockSpec((H, D2), lambda i: (0, 0)),
        pl.BlockSpec((1, H), lambda i: (0, 0)),
    ],
    out_specs=pl.BlockSpec((TB, H), lambda i: (i, 0)),
    out_shape=jax.ShapeDtypeStruct((N, H), jnp.float32),
)


def kernel(input_ids, emb0, emb1, emb2, proj1_w, proj1_b, proj2_w, proj2_b):
    ids = input_ids.reshape(-1).astype(jnp.int32)
    g0, g1, g2 = _make_sc_gather()(ids, emb0, emb1, emb2)
    out = _tc_combine(ids.reshape(N, 1), g0, g1, g2,
                      proj1_w, proj1_b.reshape(1, H),
                      proj2_w, proj2_b.reshape(1, H))
    return out.reshape(B, S, H)


# speedup vs baseline: 1.6516x; 1.6396x over previous
import functools

import jax
import jax.numpy as jnp
from jax import lax
from jax.experimental import pallas as pl
from jax.experimental.pallas import tpu as pltpu
from jax.experimental.pallas import tpu_sc as plsc

B, S, H = 4, 4096, 2048
N = B * S
D1, D2 = 512, 128
CUT0, CUT1 = 25000, 50000
SZ1, SZ2 = 25000, 50000

NW = 32
TOK_PER_W = N // NW
LANES = 16

CK1 = 64
NCH1 = TOK_PER_W // CK1
CK0 = 32
NCH0 = TOK_PER_W // CK0


@functools.lru_cache(maxsize=1)
def _make_sc_gather12():
    mesh = plsc.VectorSubcoreMesh(core_axis_name="c", subcore_axis_name="s")

    @functools.partial(
        pl.kernel,
        mesh=mesh,
        compiler_params=pltpu.CompilerParams(needs_layout_passes=False),
        out_type=(
            jax.ShapeDtypeStruct((N, D1), jnp.float32),
            jax.ShapeDtypeStruct((N, D2), jnp.float32),
        ),
        scratch_types=[
            pltpu.VMEM((TOK_PER_W,), jnp.int32),
            pltpu.VMEM((TOK_PER_W,), jnp.int32),
            pltpu.VMEM((TOK_PER_W,), jnp.int32),
            pltpu.VMEM((2, CK1, D1), jnp.float32),
            pltpu.VMEM((2, CK1, D2), jnp.float32),
            pltpu.SemaphoreType.DMA,
            pltpu.SemaphoreType.DMA,
        ],
    )
    def _sc_gather12(ids_hbm, emb1_hbm, emb2_hbm, g1_hbm, g2_hbm,
                     ids_v, idx1_v, idx2_v, buf1, buf2, gsem, wsem):
        wid = lax.axis_index("s") * 2 + lax.axis_index("c")
        base = wid * TOK_PER_W
        pltpu.sync_copy(ids_hbm.at[pl.ds(base, TOK_PER_W)], ids_v)
        for i in range(TOK_PER_W // LANES):
            v = ids_v[pl.ds(i * LANES, LANES)]
            idx1_v[pl.ds(i * LANES, LANES)] = jnp.clip(v - CUT0, 0, SZ1 - 1)
            idx2_v[pl.ds(i * LANES, LANES)] = jnp.clip(v - CUT1, 0, SZ2 - 1)

        def issue_gather(c):
            b = c % 2
            return (
                pltpu.async_copy(
                    emb1_hbm.at[idx1_v.at[pl.ds(c * CK1, CK1)]], buf1.at[b], gsem),
                pltpu.async_copy(
                    emb2_hbm.at[idx2_v.at[pl.ds(c * CK1, CK1)]], buf2.at[b], gsem),
            )

        g_cur = issue_gather(0)
        wb_hist = []
        for c in range(NCH1):
            g_cur[0].wait()
            g_cur[1].wait()
            b = c % 2
            off = base + c * CK1
            wb_hist.append((
                pltpu.async_copy(buf1.at[b], g1_hbm.at[pl.ds(off, CK1)], wsem),
                pltpu.async_copy(buf2.at[b], g2_hbm.at[pl.ds(off, CK1)], wsem),
            ))
            if c + 1 < NCH1:
                if len(wb_hist) >= 2:
                    old = wb_hist[-2]
                    old[0].wait()
                    old[1].wait()
                g_cur = issue_gather(c + 1)
        for wb in wb_hist[-2:]:
            wb[0].wait()
            wb[1].wait()

    return _sc_gather12


@functools.lru_cache(maxsize=1)
def _make_sc_overwrite0():
    mesh = plsc.VectorSubcoreMesh(core_axis_name="c", subcore_axis_name="s")

    @functools.partial(
        pl.kernel,
        mesh=mesh,
        compiler_params=pltpu.CompilerParams(needs_layout_passes=False),
        out_type=(),
        scratch_types=[
            pltpu.VMEM((TOK_PER_W,), jnp.int32),
            pltpu.VMEM((NCH0, CK0), jnp.int32),
            pltpu.VMEM((TOK_PER_W,), jnp.int32),
            pltpu.VMEM((CK0, H), jnp.float32),
            pltpu.SemaphoreType.DMA,
        ],
    )
    def _sc_overwrite0(ids_hbm, emb0_hbm, p_ref,
                       ids_v, pos2d_v, id0_v, buf0, sem):
        wid = lax.axis_index("s") * 2 + lax.axis_index("c")
        base = wid * TOK_PER_W
        pltpu.sync_copy(ids_hbm.at[pl.ds(base, TOK_PER_W)], ids_v)
        lane = lax.iota(jnp.int32, LANES)
        count = jnp.int32(0)
        for i in range(TOK_PER_W // LANES):
            v = ids_v[pl.ds(i * LANES, LANES)]
            m = v < CUT0
            mi = jnp.where(m, jnp.int32(1), jnp.int32(0))
            offs = count + plsc.cumsum(mi) - mi
            pos = base + i * LANES + lane
            plsc.store_scatter(pos2d_v, [offs >> 5, offs & 31], pos, mask=m)
            plsc.store_scatter(id0_v, [offs], v, mask=m)
            count = count + jnp.sum(mi)

        @pl.when(count > 0)
        def _():
            nch = (count + CK0 - 1) // CK0
            padded = nch * CK0
            zeros = jnp.zeros((LANES,), jnp.int32)
            pos0 = plsc.load_gather(pos2d_v, [zeros, zeros])
            id0 = plsc.load_gather(id0_v, [zeros])
            for j in range(2):
                idx = count + j * LANES + lane
                mj = idx < padded
                plsc.store_scatter(pos2d_v, [idx >> 5, idx & 31], pos0, mask=mj)
                plsc.store_scatter(id0_v, [idx], id0, mask=mj)

            def body(c, _):
                pltpu.async_copy(
                    emb0_hbm.at[id0_v.at[pl.ds(c * CK0, CK0)]], buf0, sem
                ).wait()
                pltpu.async_copy(buf0, p_ref.at[pos2d_v.at[c]], sem).wait()
                return 0

            lax.fori_loop(0, nch, body, 0)

    return _sc_overwrite0


TB = 256


def _tc_body(ids_ref, g1_ref, g2_ref, w1_ref, b1_ref, w2_ref, b2_ref, out_ref):
    ids = ids_ref[...]
    p1 = lax.dot_general(g1_ref[...], w1_ref[...], (((1,), (1,)), ((), ())),
                         preferred_element_type=jnp.float32) + b1_ref[...]
    p2 = lax.dot_general(g2_ref[...], w2_ref[...], (((1,), (1,)), ((), ())),
                         preferred_element_type=jnp.float32) + b2_ref[...]
    out_ref[...] = jnp.where(ids < CUT1, p1, p2)


_tc_combine = pl.pallas_call(
    _tc_body,
    grid=(N // TB,),
    in_specs=[
        pl.BlockSpec((TB, 1), lambda i: (i, 0)),
        pl.BlockSpec((TB, D1), lambda i: (i, 0)),
        pl.BlockSpec((TB, D2), lambda i: (i, 0)),
        pl.BlockSpec((H, D1), lambda i: (0, 0)),
        pl.BlockSpec((1, H), lambda i: (0, 0)),
        pl.BlockSpec((H, D2), lambda i: (0, 0)),
        pl.BlockSpec((1, H), lambda i: (0, 0)),
    ],
    out_specs=pl.BlockSpec((TB, H), lambda i: (i, 0)),
    out_shape=jax.ShapeDtypeStruct((N, H), jnp.float32),
)


def kernel(input_ids, emb0, emb1, emb2, proj1_w, proj1_b, proj2_w, proj2_b):
    ids = input_ids.reshape(-1).astype(jnp.int32)
    g1, g2 = _make_sc_gather12()(ids, emb1, emb2)
    p = _tc_combine(ids.reshape(N, 1), g1, g2,
                    proj1_w, proj1_b.reshape(1, H),
                    proj2_w, proj2_b.reshape(1, H))
    p_ref = jax.new_ref(p)
    _make_sc_overwrite0()(ids, emb0, p_ref)
    return p_ref[...].reshape(B, S, H)
